# baseline (speedup 1.0000x reference)
import functools

import jax
import jax.numpy as jnp
from jax import lax
from jax.experimental import pallas as pl
from jax.experimental.pallas import tpu as pltpu
from jax.experimental.pallas import tpu_sc as plsc

_B = 16384
_EMB = 128
_NDENSE = 13
_HID = 64

_NC = 2
_NS = 16
_NW = _NC * _NS
_HALVES = 2
_BH = _B // _HALVES
_BPW = _BH // _NW
_CHUNK = 128
_NCH = _BPW // _CHUNK

_BLK = 4096


def _gather_body(table_hbm, idx_hbm, out_hbm, idx_v, rows_v, gsem, wsem):
    wid = lax.axis_index("s") * _NC + lax.axis_index("c")
    pltpu.sync_copy(idx_hbm.at[pl.ds(wid * _NCH, _NCH)], idx_v)
    gathers = [
        pltpu.async_copy(
            table_hbm.at[idx_v.at[j]],
            rows_v.at[pl.ds(j * _CHUNK, _CHUNK)],
            gsem,
        )
        for j in range(_NCH)
    ]
    writes = []
    for j in range(_NCH):
        gathers[j].wait()
        writes.append(
            pltpu.async_copy(
                rows_v.at[pl.ds(j * _CHUNK, _CHUNK)],
                out_hbm.at[pl.ds(wid * _BPW + j * _CHUNK, _CHUNK)],
                wsem,
            )
        )
    for w in writes:
        w.wait()


def _make_gather():
    return pl.kernel(
        _gather_body,
        out_type=jax.ShapeDtypeStruct((_BH, _EMB), jnp.float32),
        scratch_types=[
            pltpu.VMEM((_NCH, _CHUNK), jnp.int32),
            pltpu.VMEM((_BPW, _EMB), jnp.float32),
            pltpu.SemaphoreType.DMA,
            pltpu.SemaphoreType.DMA,
        ],
        mesh=plsc.VectorSubcoreMesh(core_axis_name="c", subcore_axis_name="s"),
    )


def _mlp_body(e_ref, xdt_ref, w1_ref, b1_ref, w2_ref, b2_ref,
              w3_ref, b3_ref, out_ref):
    f32 = jnp.float32
    h1 = lax.dot_general(e_ref[...], w1_ref[:, :_EMB],
                         (((1,), (1,)), ((), ())),
                         preferred_element_type=f32)
    h1 = h1 + lax.dot_general(xdt_ref[...], w1_ref[:, _EMB:],
                              (((0,), (1,)), ((), ())),
                              preferred_element_type=f32)
    h1 = jnp.maximum(h1 + b1_ref[...], 0.0)
    h2 = lax.dot_general(h1, w2_ref[...], (((1,), (1,)), ((), ())),
                         preferred_element_type=f32)
    h2 = jnp.maximum(h2 + b2_ref[...], 0.0)
    out_ref[...] = (
        lax.dot_general(w3_ref[...], h2, (((1,), (1,)), ((), ())),
                        preferred_element_type=f32)
        + b3_ref[...]
    )


def _mlp(e, xd_t, w1, b1, w2, b2, w3, b3):
    n_blk = _BH // _BLK
    full = lambda shape: pl.BlockSpec(shape, lambda i: (0, 0))
    return pl.pallas_call(
        _mlp_body,
        grid=(n_blk,),
        in_specs=[
            pl.BlockSpec((_BLK, _EMB), lambda i: (i, 0)),
            pl.BlockSpec((_NDENSE, _BLK), lambda i: (0, i)),
            full((_HID, _EMB + _NDENSE)),
            full((1, _HID)),
            full((_HID, _HID)),
            full((1, _HID)),
            full((1, _HID)),
            full((1, 1)),
        ],
        out_specs=pl.BlockSpec((1, _BLK), lambda i: (0, i)),
        out_shape=jax.ShapeDtypeStruct((1, _BH), jnp.float32),
    )(e, xd_t, w1, b1, w2, b2, w3, b3)


def kernel(X_emb, X_dense, I_W, W1, b1, W2, b2, W3, b3):
    idx = X_emb.astype(jnp.int32)
    xd_t = X_dense.T
    b1r, b2r, b3r = b1.reshape(1, _HID), b2.reshape(1, _HID), b3.reshape(1, 1)
    gather = _make_gather()
    halves = []
    for h in range(_HALVES):
        idx2d = idx[h * _BH:(h + 1) * _BH].reshape(_NW * _NCH, _CHUNK)
        e = gather(I_W, idx2d)
        halves.append(
            _mlp(e, xd_t[:, h * _BH:(h + 1) * _BH],
                 W1, b1r, W2, b2r, W3, b3r))
    return jnp.concatenate(halves, axis=1).T


# speedup vs baseline: 1.0513x; 1.0513x over previous
import functools

import jax
import jax.numpy as jnp
from jax import lax
from jax.experimental import pallas as pl
from jax.experimental.pallas import tpu as pltpu
from jax.experimental.pallas import tpu_sc as plsc

_B = 16384
_EMB = 128
_NDENSE = 13
_HID = 64

_NC = 2
_NS = 16
_NW = _NC * _NS
_HALVES = 1
_BH = _B // _HALVES
_BPW = _BH // _NW
_CHUNK = 128
_NCH = _BPW // _CHUNK

_BLK = 2048


def _gather_body(table_hbm, idx_hbm, out_hbm, idx_v, rows_v, gsem, wsem):
    wid = lax.axis_index("s") * _NC + lax.axis_index("c")
    pltpu.sync_copy(idx_hbm.at[pl.ds(wid * _NCH, _NCH)], idx_v)
    gathers = [
        pltpu.async_copy(
            table_hbm.at[idx_v.at[j]],
            rows_v.at[pl.ds(j * _CHUNK, _CHUNK)],
            gsem,
        )
        for j in range(_NCH)
    ]
    writes = []
    for j in range(_NCH):
        gathers[j].wait()
        writes.append(
            pltpu.async_copy(
                rows_v.at[pl.ds(j * _CHUNK, _CHUNK)],
                out_hbm.at[pl.ds(wid * _BPW + j * _CHUNK, _CHUNK)],
                wsem,
            )
        )
    for w in writes:
        w.wait()


def _make_gather():
    return pl.kernel(
        _gather_body,
        out_type=jax.ShapeDtypeStruct((_BH, _EMB), jnp.float32),
        scratch_types=[
            pltpu.VMEM((_NCH, _CHUNK), jnp.int32),
            pltpu.VMEM((_BPW, _EMB), jnp.float32),
            pltpu.SemaphoreType.DMA,
            pltpu.SemaphoreType.DMA,
        ],
        mesh=plsc.VectorSubcoreMesh(core_axis_name="c", subcore_axis_name="s"),
    )


def _mlp_body(e_ref, xdt_ref, w1_ref, b1_ref, w2_ref, b2_ref,
              w3_ref, b3_ref, out_ref):
    f32 = jnp.float32
    h1 = lax.dot_general(e_ref[...], w1_ref[:, :_EMB],
                         (((1,), (1,)), ((), ())),
                         preferred_element_type=f32)
    h1 = h1 + lax.dot_general(xdt_ref[...], w1_ref[:, _EMB:],
                              (((0,), (1,)), ((), ())),
                              preferred_element_type=f32)
    h1 = jnp.maximum(h1 + b1_ref[...], 0.0)
    h2 = lax.dot_general(h1, w2_ref[...], (((1,), (1,)), ((), ())),
                         preferred_element_type=f32)
    h2 = jnp.maximum(h2 + b2_ref[...], 0.0)
    out_ref[...] = (
        lax.dot_general(w3_ref[...], h2, (((1,), (1,)), ((), ())),
                        preferred_element_type=f32)
        + b3_ref[...]
    )


def _mlp(e, xd_t, w1, b1, w2, b2, w3, b3):
    n_blk = _BH // _BLK
    full = lambda shape: pl.BlockSpec(shape, lambda i: (0, 0))
    return pl.pallas_call(
        _mlp_body,
        grid=(n_blk,),
        in_specs=[
            pl.BlockSpec((_BLK, _EMB), lambda i: (i, 0)),
            pl.BlockSpec((_NDENSE, _BLK), lambda i: (0, i)),
            full((_HID, _EMB + _NDENSE)),
            full((1, _HID)),
            full((_HID, _HID)),
            full((1, _HID)),
            full((1, _HID)),
            full((1, 1)),
        ],
        out_specs=pl.BlockSpec((1, _BLK), lambda i: (0, i)),
        out_shape=jax.ShapeDtypeStruct((1, _BH), jnp.float32),
    )(e, xd_t, w1, b1, w2, b2, w3, b3)


def kernel(X_emb, X_dense, I_W, W1, b1, W2, b2, W3, b3):
    idx = X_emb.astype(jnp.int32)
    xd_t = X_dense.T
    b1r, b2r, b3r = b1.reshape(1, _HID), b2.reshape(1, _HID), b3.reshape(1, 1)
    gather = _make_gather()
    halves = []
    for h in range(_HALVES):
        idx2d = idx[h * _BH:(h + 1) * _BH].reshape(_NW * _NCH, _CHUNK)
        e = gather(I_W, idx2d)
        halves.append(
            _mlp(e, xd_t[:, h * _BH:(h + 1) * _BH],
                 W1, b1r, W2, b2r, W3, b3r))
    return jnp.concatenate(halves, axis=1).T


# speedup vs baseline: 1.0865x; 1.0334x over previous
import functools

import jax
import jax.numpy as jnp
from jax import lax
from jax.experimental import pallas as pl
from jax.experimental.pallas import tpu as pltpu
from jax.experimental.pallas import tpu_sc as plsc

_B = 16384
_EMB = 128
_NDENSE = 13
_HID = 64

_NC = 2
_NS = 16
_NW = _NC * _NS
_HALVES = 1
_BH = _B // _HALVES
_BPW = _BH // _NW
_CHUNK = 128
_NCH = _BPW // _CHUNK

_BLK = 8192


def _gather_body(table_hbm, idx_hbm, out_hbm, idx_v, rows_v, gsem, wsem):
    wid = lax.axis_index("s") * _NC + lax.axis_index("c")
    pltpu.sync_copy(idx_hbm.at[pl.ds(wid * _NCH, _NCH)], idx_v)
    gathers = [
        pltpu.async_copy(
            table_hbm.at[idx_v.at[j]],
            rows_v.at[pl.ds(j * _CHUNK, _CHUNK)],
            gsem,
        )
        for j in range(_NCH)
    ]
    writes = []
    for j in range(_NCH):
        gathers[j].wait()
        writes.append(
            pltpu.async_copy(
                rows_v.at[pl.ds(j * _CHUNK, _CHUNK)],
                out_hbm.at[pl.ds(wid * _BPW + j * _CHUNK, _CHUNK)],
                wsem,
            )
        )
    for w in writes:
        w.wait()


def _make_gather():
    return pl.kernel(
        _gather_body,
        out_type=jax.ShapeDtypeStruct((_BH, _EMB), jnp.float32),
        scratch_types=[
            pltpu.VMEM((_NCH, _CHUNK), jnp.int32),
            pltpu.VMEM((_BPW, _EMB), jnp.float32),
            pltpu.SemaphoreType.DMA,
            pltpu.SemaphoreType.DMA,
        ],
        mesh=plsc.VectorSubcoreMesh(core_axis_name="c", subcore_axis_name="s"),
    )


def _mlp_body(e_ref, xdt_ref, w1_ref, b1_ref, w2_ref, b2_ref,
              w3_ref, b3_ref, out_ref):
    f32 = jnp.float32
    h1 = lax.dot_general(e_ref[...], w1_ref[:, :_EMB],
                         (((1,), (1,)), ((), ())),
                         preferred_element_type=f32)
    h1 = h1 + lax.dot_general(xdt_ref[...], w1_ref[:, _EMB:],
                              (((0,), (1,)), ((), ())),
                              preferred_element_type=f32)
    h1 = jnp.maximum(h1 + b1_ref[...], 0.0)
    h2 = lax.dot_general(h1, w2_ref[...], (((1,), (1,)), ((), ())),
                         preferred_element_type=f32)
    h2 = jnp.maximum(h2 + b2_ref[...], 0.0)
    out_ref[...] = (
        lax.dot_general(w3_ref[...], h2, (((1,), (1,)), ((), ())),
                        preferred_element_type=f32)
        + b3_ref[...]
    )


def _mlp(e, xd_t, w1, b1, w2, b2, w3, b3):
    n_blk = _BH // _BLK
    full = lambda shape: pl.BlockSpec(shape, lambda i: (0, 0))
    return pl.pallas_call(
        _mlp_body,
        grid=(n_blk,),
        in_specs=[
            pl.BlockSpec((_BLK, _EMB), lambda i: (i, 0)),
            pl.BlockSpec((_NDENSE, _BLK), lambda i: (0, i)),
            full((_HID, _EMB + _NDENSE)),
            full((1, _HID)),
            full((_HID, _HID)),
            full((1, _HID)),
            full((1, _HID)),
            full((1, 1)),
        ],
        out_specs=pl.BlockSpec((1, _BLK), lambda i: (0, i)),
        out_shape=jax.ShapeDtypeStruct((1, _BH), jnp.float32),
    )(e, xd_t, w1, b1, w2, b2, w3, b3)


def kernel(X_emb, X_dense, I_W, W1, b1, W2, b2, W3, b3):
    idx = X_emb.astype(jnp.int32)
    xd_t = X_dense.T
    b1r, b2r, b3r = b1.reshape(1, _HID), b2.reshape(1, _HID), b3.reshape(1, 1)
    gather = _make_gather()
    halves = []
    for h in range(_HALVES):
        idx2d = idx[h * _BH:(h + 1) * _BH].reshape(_NW * _NCH, _CHUNK)
        e = gather(I_W, idx2d)
        halves.append(
            _mlp(e, xd_t[:, h * _BH:(h + 1) * _BH],
                 W1, b1r, W2, b2r, W3, b3r))
    return jnp.concatenate(halves, axis=1).T
